# Initial kernel scaffold; baseline (speedup 1.0000x reference)
import jax
import jax.numpy as jnp
from jax.experimental import pallas as pl


def kernel(x, edge_index, edge_attr, g1, b1, g2, b2, We1a, be1a, We1b, be1b, W1a, bb1a, W1b, bb1b, We2a, be2a, We2b, be2b, W2a, bb2a, W2b, bb2b):
    raise NotImplementedError("write your pallas kernel here")


# speedup vs baseline: 2.5618x; 2.5618x over previous
import functools

import jax
import jax.numpy as jnp
from jax import lax
from jax.experimental import pallas as pl
from jax.experimental.pallas import tpu as pltpu
from jax.experimental.pallas import tpu_sc as plsc

N = 10000
E = 320000
D = 128
DH = 8
DE = 16

NC = 2
NS = 16
NW = NC * NS
LANES = 16

EDGES_PER_TILE = E // NW
CHUNK = 80
NCHUNKS = EDGES_PER_TILE // CHUNK
NPAD = 10240
ROWS_PER_TILE = NPAD // NS
ZROWS = 128


def _leaky(x):
    return jnp.where(x > 0, x, 0.01 * x)


def _bn_leaky_body(x_ref, g_ref, b_ref, o_ref):
    x = x_ref[...]
    m = jnp.mean(x, axis=0, keepdims=True)
    v = jnp.mean((x - m) ** 2, axis=0, keepdims=True)
    o_ref[...] = _leaky((x - m) / jnp.sqrt(v + 1e-5) * g_ref[...] + b_ref[...])


def _bn_leaky(x, g, b):
    return pl.pallas_call(
        _bn_leaky_body,
        out_shape=jax.ShapeDtypeStruct((N, D), jnp.float32),
    )(x, g.reshape(1, D), b.reshape(1, D))


BLK_E = 3200


def _edge_w_body(ea_ref, wa_ref, ba_ref, wb_ref, bb_ref, o_ref):
    h = _leaky(
        jnp.dot(ea_ref[...], wa_ref[...], preferred_element_type=jnp.float32)
        + ba_ref[...]
    )
    z = (
        jnp.dot(h, wb_ref[...], preferred_element_type=jnp.float32)
        + bb_ref[...]
    )
    o_ref[...] = jnp.where(z > 0, z, jnp.exp(z) - 1.0)


def _edge_weights(edge_attr, Wea, bea, Web, beb):
    grid = (E // BLK_E,)
    return pl.pallas_call(
        _edge_w_body,
        grid=grid,
        in_specs=[
            pl.BlockSpec((BLK_E, DE), lambda i: (i, 0)),
            pl.BlockSpec((DE, DH), lambda i: (0, 0)),
            pl.BlockSpec((1, DH), lambda i: (0, 0)),
            pl.BlockSpec((DH, D), lambda i: (0, 0)),
            pl.BlockSpec((1, D), lambda i: (0, 0)),
        ],
        out_specs=pl.BlockSpec((BLK_E, D), lambda i: (i, 0)),
        out_shape=jax.ShapeDtypeStruct((E, D), jnp.float32),
    )(edge_attr, Wea.T, bea.reshape(1, DH), Web.T, beb.reshape(1, D))


def _mlp_bn_body(p_ref, h_ref, wa_ref, ba_ref, wb_ref, bb_ref, g_ref, b_ref,
                 o_ref):
    s = p_ref[:N, :] + p_ref[NPAD:NPAD + N, :] + h_ref[...]
    t = _leaky(
        jnp.dot(s, wa_ref[...], preferred_element_type=jnp.float32)
        + ba_ref[...]
    )
    u = (
        jnp.dot(t, wb_ref[...], preferred_element_type=jnp.float32)
        + bb_ref[...]
    )
    m = jnp.mean(u, axis=0, keepdims=True)
    v = jnp.mean((u - m) ** 2, axis=0, keepdims=True)
    o_ref[...] = _leaky((u - m) / jnp.sqrt(v + 1e-5) * g_ref[...] + b_ref[...])


def _mlp_bn(p, h, Wa, ba, Wb, bb, g, b):
    return pl.pallas_call(
        _mlp_bn_body,
        out_shape=jax.ShapeDtypeStruct((N, D), jnp.float32),
    )(p, h, Wa.T, ba.reshape(1, D), Wb.T, bb.reshape(1, D),
      g.reshape(1, D), b.reshape(1, D))


def _mlp_skip_body(p_ref, h_ref, wa_ref, ba_ref, wb_ref, bb_ref, skip_ref,
                   o_ref):
    s = p_ref[:N, :] + p_ref[NPAD:NPAD + N, :] + h_ref[...]
    t = _leaky(
        jnp.dot(s, wa_ref[...], preferred_element_type=jnp.float32)
        + ba_ref[...]
    )
    o_ref[...] = (
        jnp.dot(t, wb_ref[...], preferred_element_type=jnp.float32)
        + bb_ref[...]
        + skip_ref[...]
    )


def _mlp_skip(p, h, Wa, ba, Wb, bb, skip):
    return pl.pallas_call(
        _mlp_skip_body,
        out_shape=jax.ShapeDtypeStruct((N, D), jnp.float32),
    )(p, h, Wa.T, ba.reshape(1, D), Wb.T, bb.reshape(1, D), skip)


def _sc_body(h_hbm, w_hbm, src_hbm, dst_hbm, out_hbm,
             src_v, dst_v, h_rows, w_rows, zbuf, agg_sh, sem):
    core = lax.axis_index("c")
    sid = lax.axis_index("s")
    wid = sid * NC + core

    def zero_row(r, _):
        for j in range(D // LANES):
            zbuf[r, pl.ds(j * LANES, LANES)] = jnp.zeros(
                (LANES,), jnp.float32)
        return 0

    lax.fori_loop(0, ZROWS, zero_row, 0)
    for t in range(ROWS_PER_TILE // ZROWS):
        pltpu.sync_copy(
            zbuf, agg_sh.at[pl.ds(sid * ROWS_PER_TILE + t * ZROWS, ZROWS), :])
    plsc.subcore_barrier()

    base_e = wid * EDGES_PER_TILE

    def chunk_body(k, _):
        off = base_e + k * CHUNK
        pltpu.sync_copy(src_hbm.at[pl.ds(off, CHUNK)], src_v)
        pltpu.sync_copy(dst_hbm.at[pl.ds(off, CHUNK)], dst_v)
        pltpu.async_copy(h_hbm.at[src_v], h_rows, sem).wait()
        pltpu.sync_copy(w_hbm.at[pl.ds(off, CHUNK), :], w_rows)

        def mul_row(e, _):
            for j in range(D // LANES):
                sl = pl.ds(j * LANES, LANES)
                w_rows[e, sl] = w_rows[e, sl] * h_rows[e, sl]
            return 0

        lax.fori_loop(0, CHUNK, mul_row, 0)
        pltpu.sync_copy(w_rows, agg_sh.at[dst_v], add=True)
        return 0

    lax.fori_loop(0, NCHUNKS, chunk_body, 0)
    plsc.subcore_barrier()

    pltpu.sync_copy(
        agg_sh.at[pl.ds(sid * ROWS_PER_TILE, ROWS_PER_TILE), :],
        out_hbm.at[pl.ds(core * NPAD + sid * ROWS_PER_TILE, ROWS_PER_TILE), :])


_sc_msgpass = pl.kernel(
    _sc_body,
    out_type=jax.ShapeDtypeStruct((NC * NPAD, D), jnp.float32),
    mesh=plsc.VectorSubcoreMesh(core_axis_name="c", subcore_axis_name="s"),
    scratch_types=[
        pltpu.VMEM((CHUNK,), jnp.int32),
        pltpu.VMEM((CHUNK,), jnp.int32),
        pltpu.VMEM((CHUNK, D), jnp.float32),
        pltpu.VMEM((CHUNK, D), jnp.float32),
        pltpu.VMEM((ZROWS, D), jnp.float32),
        pltpu.VMEM_SHARED((NPAD, D), jnp.float32),
        pltpu.SemaphoreType.DMA,
    ],
)


def kernel(x, edge_index, edge_attr, g1, b1, g2, b2, We1a, be1a, We1b, be1b,
           W1a, bb1a, W1b, bb1b, We2a, be2a, We2b, be2b, W2a, bb2a, W2b,
           bb2b):
    src = edge_index[0]
    dst = edge_index[1]

    h1 = _bn_leaky(x, g1, b1)
    w1 = _edge_weights(edge_attr, We1a, be1a, We1b, be1b)
    p1 = _sc_msgpass(h1, w1, src, dst)
    h2 = _mlp_bn(p1, h1, W1a, bb1a, W1b, bb1b, g2, b2)
    w2 = _edge_weights(edge_attr, We2a, be2a, We2b, be2b)
    p2 = _sc_msgpass(h2, w2, src, dst)
    return _mlp_skip(p2, h2, W2a, bb2a, W2b, bb2b, x)
